# Initial kernel scaffold; baseline (speedup 1.0000x reference)
import jax
import jax.numpy as jnp
from jax.experimental import pallas as pl


def kernel(x, edge_index, Wl0, bl0, Wr0, Wl1, bl1, Wr1, Wl2, bl2, Wr2):
    raise NotImplementedError("write your pallas kernel here")


# speedup vs baseline: 2.3639x; 2.3639x over previous
import functools

import jax
import jax.numpy as jnp
from jax import lax
from jax.experimental import pallas as pl
from jax.experimental.pallas import tpu as pltpu
from jax.experimental.pallas import tpu_sc as plsc

N_NODES = 10000
N_EDGES = 160000
NPAD = 10240
DUMMY = N_NODES
NSC = 2
NTILE = 16
NBATCH = 40
BEDGE = 128
EPAD = NSC * NTILE * NBATCH * BEDGE
ROWS_PER_TILE = NPAD // NTILE
BM = 256


def _make_sc_agg(nchk):
  mesh = plsc.VectorSubcoreMesh(core_axis_name="c", subcore_axis_name="s")

  out_type = tuple(
      jax.ShapeDtypeStruct((NSC, NPAD, 128), jnp.float32) for _ in range(nchk))
  scratch = [
      pltpu.VMEM((NBATCH, BEDGE), jnp.int32),
      pltpu.VMEM((NBATCH, BEDGE), jnp.int32),
      pltpu.VMEM((BEDGE, 128), jnp.float32),
      pltpu.VMEM_SHARED((NPAD, 128), jnp.float32),
      pltpu.SemaphoreType.DMA,
  ]

  @functools.partial(pl.kernel, mesh=mesh, out_type=out_type,
                     scratch_types=scratch)
  def k(*refs):
    vals = refs[:nchk]
    srcp, dstp, zeros_h = refs[nchk:nchk + 3]
    outs = refs[nchk + 3:nchk + 3 + nchk]
    src_v, dst_v, rows_v, acc, sem = refs[nchk + 3 + nchk:]

    c = lax.axis_index("c")
    s = lax.axis_index("s")
    row0 = s * ROWS_PER_TILE

    pltpu.sync_copy(srcp.at[c, s], src_v)
    pltpu.sync_copy(dstp.at[c, s], dst_v)

    for ck in range(nchk):
      for kk in range(ROWS_PER_TILE // 128):
        pltpu.sync_copy(zeros_h, acc.at[pl.ds(row0 + kk * 128, 128)])
      plsc.subcore_barrier()

      def batch(j, carry):
        pltpu.async_copy(vals[ck].at[src_v.at[j]], rows_v, sem).wait()
        pltpu.sync_copy(rows_v, acc.at[dst_v.at[j]], add=True)
        return carry

      lax.fori_loop(0, NBATCH, batch, 0)
      plsc.subcore_barrier()

      pltpu.sync_copy(acc.at[pl.ds(row0, ROWS_PER_TILE)],
                      outs[ck].at[c, pl.ds(row0, ROWS_PER_TILE)])
      plsc.subcore_barrier()

  return k


def _make_sc_deg():
  mesh = plsc.VectorSubcoreMesh(core_axis_name="c", subcore_axis_name="s")

  out_type = jax.ShapeDtypeStruct((NSC, NPAD, 128), jnp.float32)
  scratch = [
      pltpu.VMEM((NBATCH, BEDGE), jnp.int32),
      pltpu.VMEM((BEDGE, 128), jnp.float32),
      pltpu.VMEM_SHARED((NPAD, 128), jnp.float32),
  ]

  @functools.partial(pl.kernel, mesh=mesh, out_type=out_type,
                     scratch_types=scratch)
  def k(dstp, ones_h, zeros_h, dout, dst_v, ones_v, dacc):
    c = lax.axis_index("c")
    s = lax.axis_index("s")
    row0 = s * ROWS_PER_TILE

    pltpu.sync_copy(dstp.at[c, s], dst_v)
    pltpu.sync_copy(ones_h, ones_v)
    for kk in range(ROWS_PER_TILE // 128):
      pltpu.sync_copy(zeros_h, dacc.at[pl.ds(row0 + kk * 128, 128)])
    plsc.subcore_barrier()

    def batch(j, carry):
      pltpu.sync_copy(ones_v, dacc.at[dst_v.at[j]], add=True)
      return carry

    lax.fori_loop(0, NBATCH, batch, 0)
    plsc.subcore_barrier()

    pltpu.sync_copy(dacc.at[pl.ds(row0, ROWS_PER_TILE)],
                    dout.at[c, pl.ds(row0, ROWS_PER_TILE)])

  return k


def _make_tc_layer(nchk_in, dout, relu):
  nchk_out = dout // 128
  grid = (NPAD // BM,)
  din = nchk_in * 128

  def body(p_ref, pd_ref, x_ref, wl_ref, bl_ref, wr_ref, o_ref):
    deg = pd_ref[0, :, 0:1] + pd_ref[1, :, 0:1]
    inv = 1.0 / jnp.maximum(deg, 1.0)
    x = jnp.concatenate([x_ref[cc] for cc in range(nchk_in)], axis=-1)
    acc = jnp.dot(x, wr_ref[...], preferred_element_type=jnp.float32)
    acc += bl_ref[...]
    agg = jnp.concatenate(
        [p_ref[0, cc] + p_ref[1, cc] for cc in range(nchk_in)], axis=-1) * inv
    acc += jnp.dot(agg, wl_ref[...], preferred_element_type=jnp.float32)
    h = jnp.maximum(acc, 0.0) if relu else acc
    for co in range(nchk_out):
      o_ref[co] = h[:, co * 128:(co + 1) * 128]

  return pl.pallas_call(
      body,
      grid=grid,
      in_specs=[
          pl.BlockSpec((NSC, nchk_in, BM, 128), lambda i: (0, 0, i, 0)),
          pl.BlockSpec((NSC, BM, 128), lambda i: (0, i, 0)),
          pl.BlockSpec((nchk_in, BM, 128), lambda i: (0, i, 0)),
          pl.BlockSpec((din, dout), lambda i: (0, 0)),
          pl.BlockSpec((1, dout), lambda i: (0, 0)),
          pl.BlockSpec((din, dout), lambda i: (0, 0)),
      ],
      out_specs=pl.BlockSpec((nchk_out, BM, 128), lambda i: (0, i, 0)),
      out_shape=jax.ShapeDtypeStruct((nchk_out, NPAD, 128), jnp.float32),
  )


def _make_tc_pre2():
  grid = (NPAD // BM,)

  def body(x_ref, w_ref, z_ref, r_ref):
    x = jnp.concatenate([x_ref[cc] for cc in range(4)], axis=-1)
    zr = jnp.dot(x, w_ref[...], preferred_element_type=jnp.float32)
    z_ref[...] = zr[:, :128]
    r_ref[...] = zr[:, 128:]

  return pl.pallas_call(
      body,
      grid=grid,
      in_specs=[
          pl.BlockSpec((4, BM, 128), lambda i: (0, i, 0)),
          pl.BlockSpec((512, 256), lambda i: (0, 0)),
      ],
      out_specs=[
          pl.BlockSpec((BM, 128), lambda i: (i, 0)),
          pl.BlockSpec((BM, 128), lambda i: (i, 0)),
      ],
      out_shape=[
          jax.ShapeDtypeStruct((NPAD, 128), jnp.float32),
          jax.ShapeDtypeStruct((NPAD, 128), jnp.float32),
      ],
  )


def _make_tc_post2():
  grid = (NPAD // BM,)

  def body(p_ref, pd_ref, r_ref, bl_ref, o_ref):
    deg = pd_ref[0, :, 0:1] + pd_ref[1, :, 0:1]
    inv = 1.0 / jnp.maximum(deg, 1.0)
    o_ref[...] = (p_ref[0] + p_ref[1]) * inv + r_ref[...] + bl_ref[...]

  return pl.pallas_call(
      body,
      grid=grid,
      in_specs=[
          pl.BlockSpec((NSC, BM, 128), lambda i: (0, i, 0)),
          pl.BlockSpec((NSC, BM, 128), lambda i: (0, i, 0)),
          pl.BlockSpec((BM, 128), lambda i: (i, 0)),
          pl.BlockSpec((1, 128), lambda i: (0, 0)),
      ],
      out_specs=pl.BlockSpec((BM, 128), lambda i: (i, 0)),
      out_shape=jax.ShapeDtypeStruct((NPAD, 128), jnp.float32),
  )


def _chunked(a):
  npad, d = a.shape
  return a.reshape(npad, d // 128, 128).transpose(1, 0, 2)


@jax.jit
def kernel(x, edge_index, Wl0, bl0, Wr0, Wl1, bl1, Wr1, Wl2, bl2, Wr2):
  src = edge_index[0]
  dst = edge_index[1]
  srcp = jnp.concatenate(
      [src, jnp.zeros((EPAD - N_EDGES,), jnp.int32)]).reshape(
          NSC, NTILE, NBATCH, BEDGE)
  dstp = jnp.concatenate(
      [dst, jnp.full((EPAD - N_EDGES,), DUMMY, jnp.int32)]).reshape(
          NSC, NTILE, NBATCH, BEDGE)
  zeros128 = jnp.zeros((128, 128), jnp.float32)
  ones128 = jnp.ones((BEDGE, 128), jnp.float32)

  xc = _chunked(jnp.pad(x, ((0, NPAD - N_NODES), (0, 0))))

  pdeg = _make_sc_deg()(dstp, ones128, zeros128)

  p0a, p0b = _make_sc_agg(2)(xc[0], xc[1], srcp, dstp, zeros128)
  p0 = jnp.stack([p0a, p0b], axis=1)
  h1 = _make_tc_layer(2, 512, True)(p0, pdeg, xc, Wl0,
                                    bl0.reshape(1, -1), Wr0)

  p1s = _make_sc_agg(4)(h1[0], h1[1], h1[2], h1[3], srcp, dstp, zeros128)
  p1 = jnp.stack(p1s, axis=1)
  h2 = _make_tc_layer(4, 512, True)(p1, pdeg, h1, Wl1,
                                    bl1.reshape(1, -1), Wr1)

  w2 = jnp.concatenate([Wl2, Wr2], axis=1)
  z, r = _make_tc_pre2()(h2, w2)
  (p2,) = _make_sc_agg(1)(z, srcp, dstp, zeros128)
  out = _make_tc_post2()(p2, pdeg, r, bl2.reshape(1, -1))
  return out[:N_NODES]


# speedup vs baseline: 2.5315x; 1.0709x over previous
import functools

import jax
import jax.numpy as jnp
from jax import lax
from jax.experimental import pallas as pl
from jax.experimental.pallas import tpu as pltpu
from jax.experimental.pallas import tpu_sc as plsc

N_NODES = 10000
N_EDGES = 160000
NPAD = 10240
DUMMY = N_NODES
NSC = 2
NTILE = 16
NBATCH = 40
BEDGE = 128
EPAD = NSC * NTILE * NBATCH * BEDGE
ROWS_PER_TILE = NPAD // NTILE
BM = 256


def _make_sc_agg(nchk):
  mesh = plsc.VectorSubcoreMesh(core_axis_name="c", subcore_axis_name="s")

  out_type = tuple(
      jax.ShapeDtypeStruct((NSC, NPAD, 128), jnp.float32) for _ in range(nchk))
  scratch = [
      pltpu.VMEM((NBATCH, BEDGE), jnp.int32),
      pltpu.VMEM((NBATCH, BEDGE), jnp.int32),
      pltpu.VMEM((BEDGE, 128), jnp.float32),
      pltpu.VMEM((BEDGE, 128), jnp.float32),
      pltpu.VMEM_SHARED((NPAD, 128), jnp.float32),
      pltpu.SemaphoreType.DMA,
      pltpu.SemaphoreType.DMA,
  ]

  @functools.partial(pl.kernel, mesh=mesh, out_type=out_type,
                     scratch_types=scratch)
  def k(*refs):
    vals = refs[:nchk]
    srcp, dstp, zeros_h = refs[nchk:nchk + 3]
    outs = refs[nchk + 3:nchk + 3 + nchk]
    src_v, dst_v, rows0, rows1, acc, sem0, sem1 = refs[nchk + 3 + nchk:]

    c = lax.axis_index("c")
    s = lax.axis_index("s")
    row0 = s * ROWS_PER_TILE

    pltpu.sync_copy(srcp.at[c, s], src_v)
    pltpu.sync_copy(dstp.at[c, s], dst_v)

    for ck in range(nchk):
      vck = vals[ck]
      for kk in range(ROWS_PER_TILE // 128):
        pltpu.sync_copy(zeros_h, acc.at[pl.ds(row0 + kk * 128, 128)])
      plsc.subcore_barrier()

      pltpu.async_copy(vck.at[src_v.at[0]], rows0, sem0)

      def batch2(i, carry):
        j = 2 * i
        pltpu.make_async_copy(vck.at[src_v.at[j]], rows0, sem0).wait()
        pltpu.async_copy(vck.at[src_v.at[j + 1]], rows1, sem1)
        pltpu.sync_copy(rows0, acc.at[dst_v.at[j]], add=True)
        pltpu.make_async_copy(vck.at[src_v.at[j + 1]], rows1, sem1).wait()
        pltpu.async_copy(vck.at[src_v.at[j + 2]], rows0, sem0)
        pltpu.sync_copy(rows1, acc.at[dst_v.at[j + 1]], add=True)
        return carry

      lax.fori_loop(0, NBATCH // 2 - 1, batch2, 0)
      pltpu.make_async_copy(vck.at[src_v.at[NBATCH - 2]], rows0, sem0).wait()
      pltpu.async_copy(vck.at[src_v.at[NBATCH - 1]], rows1, sem1)
      pltpu.sync_copy(rows0, acc.at[dst_v.at[NBATCH - 2]], add=True)
      pltpu.make_async_copy(vck.at[src_v.at[NBATCH - 1]], rows1, sem1).wait()
      pltpu.sync_copy(rows1, acc.at[dst_v.at[NBATCH - 1]], add=True)
      plsc.subcore_barrier()

      pltpu.sync_copy(acc.at[pl.ds(row0, ROWS_PER_TILE)],
                      outs[ck].at[c, pl.ds(row0, ROWS_PER_TILE)])
      plsc.subcore_barrier()

  return k


def _make_sc_deg():
  mesh = plsc.VectorSubcoreMesh(core_axis_name="c", subcore_axis_name="s")

  out_type = jax.ShapeDtypeStruct((NSC, NPAD, 128), jnp.float32)
  scratch = [
      pltpu.VMEM((NBATCH, BEDGE), jnp.int32),
      pltpu.VMEM((BEDGE, 128), jnp.float32),
      pltpu.VMEM_SHARED((NPAD, 128), jnp.float32),
  ]

  @functools.partial(pl.kernel, mesh=mesh, out_type=out_type,
                     scratch_types=scratch)
  def k(dstp, ones_h, zeros_h, dout, dst_v, ones_v, dacc):
    c = lax.axis_index("c")
    s = lax.axis_index("s")
    row0 = s * ROWS_PER_TILE

    pltpu.sync_copy(dstp.at[c, s], dst_v)
    pltpu.sync_copy(ones_h, ones_v)
    for kk in range(ROWS_PER_TILE // 128):
      pltpu.sync_copy(zeros_h, dacc.at[pl.ds(row0 + kk * 128, 128)])
    plsc.subcore_barrier()

    def batch(j, carry):
      pltpu.sync_copy(ones_v, dacc.at[dst_v.at[j]], add=True)
      return carry

    lax.fori_loop(0, NBATCH, batch, 0)
    plsc.subcore_barrier()

    pltpu.sync_copy(dacc.at[pl.ds(row0, ROWS_PER_TILE)],
                    dout.at[c, pl.ds(row0, ROWS_PER_TILE)])

  return k


def _make_tc_layer(nchk_in, dout, relu):
  nchk_out = dout // 128
  grid = (NPAD // BM,)
  din = nchk_in * 128

  def body(p_ref, pd_ref, x_ref, wl_ref, bl_ref, wr_ref, o_ref):
    deg = pd_ref[0, :, 0:1] + pd_ref[1, :, 0:1]
    inv = 1.0 / jnp.maximum(deg, 1.0)
    x = jnp.concatenate([x_ref[cc] for cc in range(nchk_in)], axis=-1)
    acc = jnp.dot(x, wr_ref[...], preferred_element_type=jnp.float32)
    acc += bl_ref[...]
    agg = jnp.concatenate(
        [p_ref[0, cc] + p_ref[1, cc] for cc in range(nchk_in)], axis=-1) * inv
    acc += jnp.dot(agg, wl_ref[...], preferred_element_type=jnp.float32)
    h = jnp.maximum(acc, 0.0) if relu else acc
    for co in range(nchk_out):
      o_ref[co] = h[:, co * 128:(co + 1) * 128]

  return pl.pallas_call(
      body,
      grid=grid,
      in_specs=[
          pl.BlockSpec((NSC, nchk_in, BM, 128), lambda i: (0, 0, i, 0)),
          pl.BlockSpec((NSC, BM, 128), lambda i: (0, i, 0)),
          pl.BlockSpec((nchk_in, BM, 128), lambda i: (0, i, 0)),
          pl.BlockSpec((din, dout), lambda i: (0, 0)),
          pl.BlockSpec((1, dout), lambda i: (0, 0)),
          pl.BlockSpec((din, dout), lambda i: (0, 0)),
      ],
      out_specs=pl.BlockSpec((nchk_out, BM, 128), lambda i: (0, i, 0)),
      out_shape=jax.ShapeDtypeStruct((nchk_out, NPAD, 128), jnp.float32),
  )


def _make_tc_pre2():
  grid = (NPAD // BM,)

  def body(x_ref, w_ref, z_ref, r_ref):
    x = jnp.concatenate([x_ref[cc] for cc in range(4)], axis=-1)
    zr = jnp.dot(x, w_ref[...], preferred_element_type=jnp.float32)
    z_ref[...] = zr[:, :128]
    r_ref[...] = zr[:, 128:]

  return pl.pallas_call(
      body,
      grid=grid,
      in_specs=[
          pl.BlockSpec((4, BM, 128), lambda i: (0, i, 0)),
          pl.BlockSpec((512, 256), lambda i: (0, 0)),
      ],
      out_specs=[
          pl.BlockSpec((BM, 128), lambda i: (i, 0)),
          pl.BlockSpec((BM, 128), lambda i: (i, 0)),
      ],
      out_shape=[
          jax.ShapeDtypeStruct((NPAD, 128), jnp.float32),
          jax.ShapeDtypeStruct((NPAD, 128), jnp.float32),
      ],
  )


def _make_tc_post2():
  grid = (NPAD // BM,)

  def body(p_ref, pd_ref, r_ref, bl_ref, o_ref):
    deg = pd_ref[0, :, 0:1] + pd_ref[1, :, 0:1]
    inv = 1.0 / jnp.maximum(deg, 1.0)
    o_ref[...] = (p_ref[0] + p_ref[1]) * inv + r_ref[...] + bl_ref[...]

  return pl.pallas_call(
      body,
      grid=grid,
      in_specs=[
          pl.BlockSpec((NSC, BM, 128), lambda i: (0, i, 0)),
          pl.BlockSpec((NSC, BM, 128), lambda i: (0, i, 0)),
          pl.BlockSpec((BM, 128), lambda i: (i, 0)),
          pl.BlockSpec((1, 128), lambda i: (0, 0)),
      ],
      out_specs=pl.BlockSpec((BM, 128), lambda i: (i, 0)),
      out_shape=jax.ShapeDtypeStruct((NPAD, 128), jnp.float32),
  )


def _chunked(a):
  npad, d = a.shape
  return a.reshape(npad, d // 128, 128).transpose(1, 0, 2)


@jax.jit
def kernel(x, edge_index, Wl0, bl0, Wr0, Wl1, bl1, Wr1, Wl2, bl2, Wr2):
  src = edge_index[0]
  dst = edge_index[1]
  srcp = jnp.concatenate(
      [src, jnp.zeros((EPAD - N_EDGES,), jnp.int32)]).reshape(
          NSC, NTILE, NBATCH, BEDGE)
  dstp = jnp.concatenate(
      [dst, jnp.full((EPAD - N_EDGES,), DUMMY, jnp.int32)]).reshape(
          NSC, NTILE, NBATCH, BEDGE)
  zeros128 = jnp.zeros((128, 128), jnp.float32)
  ones128 = jnp.ones((BEDGE, 128), jnp.float32)

  xc = _chunked(jnp.pad(x, ((0, NPAD - N_NODES), (0, 0))))

  pdeg = _make_sc_deg()(dstp, ones128, zeros128)

  p0a, p0b = _make_sc_agg(2)(xc[0], xc[1], srcp, dstp, zeros128)
  p0 = jnp.stack([p0a, p0b], axis=1)
  h1 = _make_tc_layer(2, 512, True)(p0, pdeg, xc, Wl0,
                                    bl0.reshape(1, -1), Wr0)

  p1s = _make_sc_agg(4)(h1[0], h1[1], h1[2], h1[3], srcp, dstp, zeros128)
  p1 = jnp.stack(p1s, axis=1)
  h2 = _make_tc_layer(4, 512, True)(p1, pdeg, h1, Wl1,
                                    bl1.reshape(1, -1), Wr1)

  w2 = jnp.concatenate([Wl2, Wr2], axis=1)
  z, r = _make_tc_pre2()(h2, w2)
  (p2,) = _make_sc_agg(1)(z, srcp, dstp, zeros128)
  out = _make_tc_post2()(p2, pdeg, r, bl2.reshape(1, -1))
  return out[:N_NODES]


# speedup vs baseline: 2.7002x; 1.0666x over previous
import functools

import jax
import jax.numpy as jnp
from jax import lax
from jax.experimental import pallas as pl
from jax.experimental.pallas import tpu as pltpu
from jax.experimental.pallas import tpu_sc as plsc

N_NODES = 10000
N_EDGES = 160000
NPAD = 10240
DUMMY = N_NODES
NSC = 2
NTILE = 16
BEDGE = 128
NB0 = 56
NB1 = 24
NBTOT = NB0 + NB1
EPAD = NTILE * NBTOT * BEDGE
ROWS_PER_TILE = NPAD // NTILE
BM = 256


def _make_sc_agg(nchk, with_deg):
  mesh = plsc.VectorSubcoreMesh(core_axis_name="c", subcore_axis_name="s")

  out_type = tuple(
      jax.ShapeDtypeStruct((NSC, NPAD, 128), jnp.float32)
      for _ in range(nchk + (1 if with_deg else 0)))
  scratch = [
      pltpu.VMEM((NB0, BEDGE), jnp.int32),
      pltpu.VMEM((NB0, BEDGE), jnp.int32),
      pltpu.VMEM((BEDGE, 128), jnp.float32),
      pltpu.VMEM((BEDGE, 128), jnp.float32),
      pltpu.VMEM_SHARED((NPAD, 128), jnp.float32),
      pltpu.SemaphoreType.DMA,
      pltpu.SemaphoreType.DMA,
  ]

  @functools.partial(pl.kernel, mesh=mesh, out_type=out_type,
                     scratch_types=scratch)
  def k(*refs):
    vals = refs[:nchk]
    pos = nchk
    srcp, dstp, zeros_h = refs[pos], refs[pos + 1], refs[pos + 2]
    pos += 3
    if with_deg:
      ones_h = refs[pos]
      pos += 1
    outs = refs[pos:pos + nchk]
    pos += nchk
    if with_deg:
      dout = refs[pos]
      pos += 1
    src_v, dst_v, rows0, rows1, acc, sem0, sem1 = refs[pos:]

    c = lax.axis_index("c")
    s = lax.axis_index("s")
    row0 = s * ROWS_PER_TILE
    nb = jnp.where(c == 0, NB0, NB1)

    @pl.when(c == 0)
    def _():
      pltpu.sync_copy(srcp.at[s, pl.ds(0, NB0)], src_v)
      pltpu.sync_copy(dstp.at[s, pl.ds(0, NB0)], dst_v)

    @pl.when(c != 0)
    def _():
      pltpu.sync_copy(srcp.at[s, pl.ds(NB0, NB1)], src_v.at[pl.ds(0, NB1)])
      pltpu.sync_copy(dstp.at[s, pl.ds(NB0, NB1)], dst_v.at[pl.ds(0, NB1)])

    def zero_acc():
      for kk in range(ROWS_PER_TILE // 128):
        pltpu.sync_copy(zeros_h, acc.at[pl.ds(row0 + kk * 128, 128)])

    def flush(out):
      pltpu.sync_copy(acc.at[pl.ds(row0, ROWS_PER_TILE)],
                      out.at[c, pl.ds(row0, ROWS_PER_TILE)])

    for ck in range(nchk):
      vck = vals[ck]
      zero_acc()
      plsc.subcore_barrier()

      pltpu.async_copy(vck.at[src_v.at[0]], rows0, sem0)

      def batch2(i, carry):
        j = 2 * i
        pltpu.make_async_copy(vck.at[src_v.at[j]], rows0, sem0).wait()
        pltpu.async_copy(vck.at[src_v.at[j + 1]], rows1, sem1)
        pltpu.sync_copy(rows0, acc.at[dst_v.at[j]], add=True)
        pltpu.make_async_copy(vck.at[src_v.at[j + 1]], rows1, sem1).wait()
        pltpu.async_copy(vck.at[src_v.at[j + 2]], rows0, sem0)
        pltpu.sync_copy(rows1, acc.at[dst_v.at[j + 1]], add=True)
        return carry

      lax.fori_loop(0, nb // 2 - 1, batch2, 0)
      je = nb - 2
      pltpu.make_async_copy(vck.at[src_v.at[je]], rows0, sem0).wait()
      pltpu.async_copy(vck.at[src_v.at[je + 1]], rows1, sem1)
      pltpu.sync_copy(rows0, acc.at[dst_v.at[je]], add=True)
      pltpu.make_async_copy(vck.at[src_v.at[je + 1]], rows1, sem1).wait()
      pltpu.sync_copy(rows1, acc.at[dst_v.at[je + 1]], add=True)
      plsc.subcore_barrier()

      flush(outs[ck])
      plsc.subcore_barrier()

    if with_deg:
      pltpu.sync_copy(ones_h, rows0)
      zero_acc()
      plsc.subcore_barrier()

      def dbatch(j, carry):
        pltpu.sync_copy(rows0, acc.at[dst_v.at[j]], add=True)
        return carry

      lax.fori_loop(0, nb, dbatch, 0)
      plsc.subcore_barrier()
      flush(dout)

  return k


def _deg_inv(pd_ref):
  deg = pd_ref[0, :, 0:1] + pd_ref[1, :, 0:1]
  return 1.0 / jnp.maximum(deg, 1.0)


def _make_tc_layer0():
  grid = (NPAD // BM,)

  def body(p_ref, pd_ref, x_ref, wl_ref, bl_ref, wr_ref, o_ref):
    inv = _deg_inv(pd_ref)
    x = jnp.concatenate([x_ref[cc] for cc in range(2)], axis=-1)
    acc = jnp.dot(x, wr_ref[...], preferred_element_type=jnp.float32)
    acc += bl_ref[...]
    agg = jnp.concatenate(
        [p_ref[0, cc] + p_ref[1, cc] for cc in range(2)], axis=-1) * inv
    acc += jnp.dot(agg, wl_ref[...], preferred_element_type=jnp.float32)
    h = jnp.maximum(acc, 0.0)
    for co in range(4):
      o_ref[co] = h[:, co * 128:(co + 1) * 128]

  return pl.pallas_call(
      body,
      grid=grid,
      in_specs=[
          pl.BlockSpec((NSC, 2, BM, 128), lambda i: (0, 0, i, 0)),
          pl.BlockSpec((NSC, BM, 128), lambda i: (0, i, 0)),
          pl.BlockSpec((2, BM, 128), lambda i: (0, i, 0)),
          pl.BlockSpec((256, 512), lambda i: (0, 0)),
          pl.BlockSpec((1, 512), lambda i: (0, 0)),
          pl.BlockSpec((256, 512), lambda i: (0, 0)),
      ],
      out_specs=pl.BlockSpec((4, BM, 128), lambda i: (0, i, 0)),
      out_shape=jax.ShapeDtypeStruct((4, NPAD, 128), jnp.float32),
  )


def _make_tc_layer1():
  grid = (NPAD // BM,)

  def body(p_ref, pd_ref, x_ref, wl_ref, bl_ref, wr_ref, w2_ref,
           z_ref, r_ref):
    inv = _deg_inv(pd_ref)
    x = jnp.concatenate([x_ref[cc] for cc in range(4)], axis=-1)
    acc = jnp.dot(x, wr_ref[...], preferred_element_type=jnp.float32)
    acc += bl_ref[...]
    agg = jnp.concatenate(
        [p_ref[0, cc] + p_ref[1, cc] for cc in range(4)], axis=-1) * inv
    acc += jnp.dot(agg, wl_ref[...], preferred_element_type=jnp.float32)
    h = jnp.maximum(acc, 0.0)
    zr = jnp.dot(h, w2_ref[...], preferred_element_type=jnp.float32)
    z_ref[...] = zr[:, :128]
    r_ref[...] = zr[:, 128:]

  return pl.pallas_call(
      body,
      grid=grid,
      in_specs=[
          pl.BlockSpec((NSC, 4, BM, 128), lambda i: (0, 0, i, 0)),
          pl.BlockSpec((NSC, BM, 128), lambda i: (0, i, 0)),
          pl.BlockSpec((4, BM, 128), lambda i: (0, i, 0)),
          pl.BlockSpec((512, 512), lambda i: (0, 0)),
          pl.BlockSpec((1, 512), lambda i: (0, 0)),
          pl.BlockSpec((512, 512), lambda i: (0, 0)),
          pl.BlockSpec((512, 256), lambda i: (0, 0)),
      ],
      out_specs=[
          pl.BlockSpec((BM, 128), lambda i: (i, 0)),
          pl.BlockSpec((BM, 128), lambda i: (i, 0)),
      ],
      out_shape=[
          jax.ShapeDtypeStruct((NPAD, 128), jnp.float32),
          jax.ShapeDtypeStruct((NPAD, 128), jnp.float32),
      ],
  )


def _make_tc_post2():
  grid = (NPAD // BM,)

  def body(p_ref, pd_ref, r_ref, bl_ref, o_ref):
    inv = _deg_inv(pd_ref)
    o_ref[...] = (p_ref[0] + p_ref[1]) * inv + r_ref[...] + bl_ref[...]

  return pl.pallas_call(
      body,
      grid=grid,
      in_specs=[
          pl.BlockSpec((NSC, BM, 128), lambda i: (0, i, 0)),
          pl.BlockSpec((NSC, BM, 128), lambda i: (0, i, 0)),
          pl.BlockSpec((BM, 128), lambda i: (i, 0)),
          pl.BlockSpec((1, 128), lambda i: (0, 0)),
      ],
      out_specs=pl.BlockSpec((BM, 128), lambda i: (i, 0)),
      out_shape=jax.ShapeDtypeStruct((NPAD, 128), jnp.float32),
  )


def _chunked(a):
  npad, d = a.shape
  return a.reshape(npad, d // 128, 128).transpose(1, 0, 2)


def _edge_layout(e, fill):
  flat = jnp.concatenate([e, jnp.full((EPAD - N_EDGES,), fill, jnp.int32)])
  n0 = NTILE * NB0 * BEDGE
  e0 = flat[:n0].reshape(NTILE, NB0, BEDGE)
  e1 = flat[n0:].reshape(NTILE, NB1, BEDGE)
  return jnp.concatenate([e0, e1], axis=1)


@jax.jit
def kernel(x, edge_index, Wl0, bl0, Wr0, Wl1, bl1, Wr1, Wl2, bl2, Wr2):
  srcp = _edge_layout(edge_index[0], 0)
  dstp = _edge_layout(edge_index[1], DUMMY)
  zeros128 = jnp.zeros((128, 128), jnp.float32)
  ones128 = jnp.ones((128, 128), jnp.float32)

  xc = _chunked(jnp.pad(x, ((0, NPAD - N_NODES), (0, 0))))

  p0a, p0b, pdeg = _make_sc_agg(2, True)(xc[0], xc[1], srcp, dstp,
                                         zeros128, ones128)
  p0 = jnp.stack([p0a, p0b], axis=1)
  h1 = _make_tc_layer0()(p0, pdeg, xc, Wl0, bl0.reshape(1, -1), Wr0)

  p1s = _make_sc_agg(4, False)(h1[0], h1[1], h1[2], h1[3], srcp, dstp,
                               zeros128)
  p1 = jnp.stack(p1s, axis=1)
  w2 = jnp.concatenate([Wl2, Wr2], axis=1)
  z, r = _make_tc_layer1()(p1, pdeg, h1, Wl1, bl1.reshape(1, -1), Wr1, w2)

  (p2,) = _make_sc_agg(1, False)(z, srcp, dstp, zeros128)
  out = _make_tc_post2()(p2, pdeg, r, bl2.reshape(1, -1))
  return out[:N_NODES]
